# baseline (speedup 1.0000x reference)
import jax
import jax.numpy as jnp
from jax.experimental import pallas as pl
from jax.experimental.pallas import tpu as pltpu


def _pair_score_kernel(pos_ref, neg_ref, w1a_ref, w1b_ref, b1_ref,
                       w2_ref, consts_ref, out_ref):
    bsz = out_ref.shape[0]
    d, h = w1a_ref.shape
    p = out_ref.shape[2]
    n = neg_ref.shape[0] // bsz
    a_all = jnp.dot(pos_ref[...], w1a_ref[...],
                    preferred_element_type=jnp.float32)
    c_all = jnp.dot(neg_ref[...], w1b_ref[...],
                    preferred_element_type=jnp.float32) + b1_ref[...]
    b2 = consts_ref[0, 0]
    scale = consts_ref[0, 1]

    w2_bcast = jnp.broadcast_to(w2_ref[...], (h, p))
    w2_tiled = jnp.tile(w2_bcast, (p, 1))
    rows_q = jax.lax.broadcasted_iota(jnp.int32, (p * h, p), 0) // h
    cols = jax.lax.broadcasted_iota(jnp.int32, (p * h, p), 1)
    w2big = jnp.where(rows_q == cols, w2_tiled, 0.0)

    for b in range(bsz):
        a = a_all[b * p:(b + 1) * p]
        c = c_all[b * n:(b + 1) * n]
        a_flat = a.reshape(1, p * h)
        r = jnp.maximum(jnp.tile(c, (1, p)) + a_flat, 0.0)
        s = jnp.dot(r, w2big, preferred_element_type=jnp.float32)
        z = scale * s
        m = jnp.max(z, axis=0, keepdims=True)
        e = jnp.exp(z - m)
        out_ref[b, 0, :] = jnp.sum(e * s, axis=0) / jnp.sum(e, axis=0) + b2


def kernel(fea0, neg_fea, W1, b1, W2, b2, scale_param):
    bsz, n, d = neg_fea.shape
    pos2 = fea0.reshape(-1, d)
    neg2 = neg_fea.reshape(-1, d)
    p = fea0.size // (bsz * d)
    h = W1.shape[1]
    w1a = W1[:d]
    w1b = W1[d:]
    scale = jax.nn.softplus(scale_param)
    consts = jnp.stack([b2[0], scale]).reshape(1, 2)
    b1r = b1.reshape(1, h)

    out = pl.pallas_call(
        _pair_score_kernel,
        out_shape=jax.ShapeDtypeStruct((bsz, 1, p), jnp.float32),
    )(pos2, neg2, w1a, w1b, b1r, W2, consts)
    return out.reshape(bsz, p)


# speedup vs baseline: 1.1311x; 1.1311x over previous
import jax
import jax.numpy as jnp
from jax.experimental import pallas as pl
from jax.experimental.pallas import tpu as pltpu


def _pair_score_kernel(pos_ref, neg_ref, w1_ref, b1_ref,
                       w2_ref, sc_ref, out_ref):
    bsz = out_ref.shape[0]
    d = pos_ref.shape[1]
    h = w1_ref.shape[1]
    p = out_ref.shape[2]
    n = neg_ref.shape[0] // bsz
    w1a = w1_ref[0:d, :]
    w1b = w1_ref[d:2 * d, :]
    a_all = jnp.dot(pos_ref[...], w1a,
                    preferred_element_type=jnp.float32)
    c_all = jnp.dot(neg_ref[...], w1b,
                    preferred_element_type=jnp.float32) + b1_ref[...]
    b2 = sc_ref[0, 0]
    scale = jnp.log1p(jnp.exp(sc_ref[0, 1]))

    w2_bcast = jnp.broadcast_to(w2_ref[...], (h, p))
    w2_tiled = jnp.tile(w2_bcast, (p, 1))
    rows_q = jax.lax.broadcasted_iota(jnp.int32, (p * h, p), 0) // h
    cols = jax.lax.broadcasted_iota(jnp.int32, (p * h, p), 1)
    w2big = jnp.where(rows_q == cols, w2_tiled, 0.0)

    for b in range(bsz):
        a = a_all[b * p:(b + 1) * p]
        c = c_all[b * n:(b + 1) * n]
        a_flat = a.reshape(1, p * h)
        r = jnp.maximum(jnp.tile(c, (1, p)) + a_flat, 0.0)
        s = jnp.dot(r, w2big, preferred_element_type=jnp.float32)
        z = scale * s
        m = jnp.max(z, axis=0, keepdims=True)
        e = jnp.exp(z - m)
        out_ref[b, 0, :] = jnp.sum(e * s, axis=0) / jnp.sum(e, axis=0) + b2


def kernel(fea0, neg_fea, W1, b1, W2, b2, scale_param):
    bsz, n, d = neg_fea.shape
    pos2 = fea0.reshape(-1, d)
    neg2 = neg_fea.reshape(-1, d)
    p = fea0.size // (bsz * d)
    h = W1.shape[1]
    sc = jnp.stack([b2[0], scale_param]).reshape(1, 2)
    b1r = b1.reshape(1, h)

    out = pl.pallas_call(
        _pair_score_kernel,
        out_shape=jax.ShapeDtypeStruct((bsz, 1, p), jnp.float32),
    )(pos2, neg2, W1, b1r, W2, sc)
    return out.reshape(bsz, p)


# speedup vs baseline: 1.1802x; 1.0434x over previous
import jax
import jax.numpy as jnp
from jax.experimental import pallas as pl
from jax.experimental.pallas import tpu as pltpu


def _pair_score_kernel(pos_ref, neg_ref, w1_ref, b1_ref,
                       w2_ref, b2_ref, sp_ref, out_ref):
    bsz = out_ref.shape[0]
    d = pos_ref.shape[1]
    h = w1_ref.shape[1]
    p = out_ref.shape[2]
    n = neg_ref.shape[0] // bsz
    w1a = w1_ref[0:d, :]
    w1b = w1_ref[d:2 * d, :]
    a_all = jnp.dot(pos_ref[...], w1a,
                    preferred_element_type=jnp.float32)
    c_all = jnp.dot(neg_ref[...], w1b,
                    preferred_element_type=jnp.float32) + b1_ref[...]
    b2 = b2_ref[0, 0]
    scale = jnp.log1p(jnp.exp(sp_ref[0, 0]))

    w2_bcast = jnp.broadcast_to(w2_ref[...], (h, p))
    w2_tiled = jnp.tile(w2_bcast, (p, 1))
    rows_q = jax.lax.broadcasted_iota(jnp.int32, (p * h, p), 0) // h
    cols = jax.lax.broadcasted_iota(jnp.int32, (p * h, p), 1)
    w2big = jnp.where(rows_q == cols, w2_tiled, 0.0)

    for b in range(bsz):
        a = a_all[b * p:(b + 1) * p]
        c = c_all[b * n:(b + 1) * n]
        a_flat = a.reshape(1, p * h)
        r = jnp.maximum(jnp.tile(c, (1, p)) + a_flat, 0.0)
        s = jnp.dot(r, w2big, preferred_element_type=jnp.float32)
        z = scale * s
        m = jnp.max(z, axis=0, keepdims=True)
        e = jnp.exp(z - m)
        out_ref[b, 0, :] = jnp.sum(e * s, axis=0) / jnp.sum(e, axis=0) + b2


def kernel(fea0, neg_fea, W1, b1, W2, b2, scale_param):
    bsz, n, d = neg_fea.shape
    pos2 = fea0.reshape(-1, d)
    neg2 = neg_fea.reshape(-1, d)
    p = fea0.size // (bsz * d)
    h = W1.shape[1]
    b1r = b1.reshape(1, h)
    b2r = b2.reshape(1, 1)
    spr = scale_param.reshape(1, 1)

    out = pl.pallas_call(
        _pair_score_kernel,
        out_shape=jax.ShapeDtypeStruct((bsz, 1, p), jnp.float32),
    )(pos2, neg2, W1, b1r, W2, b2r, spr)
    return out.reshape(bsz, p)
